# baseline (speedup 1.0000x reference)
import jax
import jax.numpy as jnp
from jax.experimental import pallas as pl
from jax.experimental.pallas import tpu as pltpu

_S = 1024
_F = 512
_N = 16


def _fused_sage_body(x_ref, adj_ref, w1_ref, w2_ref, w3_ref, out_ref,
                     h_scratch):
    i = pl.program_id(0)

    @pl.when(i > 0)
    def _softmax_prev():
        h = h_scratch[...]
        m = jnp.max(h, axis=1, keepdims=True)
        lse = jnp.log(jnp.sum(jnp.exp(h - m), axis=1, keepdims=True)) + m
        out_ref[...] = h - lse

    @pl.when(i < _N)
    def _layers():
        adj = adj_ref[0].astype(jnp.bfloat16)
        h = x_ref[...]
        for j, w_ref in enumerate((w1_ref, w2_ref, w3_ref)):
            ax = jnp.dot(adj, h.astype(jnp.bfloat16),
                         preferred_element_type=jnp.float32)
            h = jax.lax.dot_general(
                (h + ax).astype(jnp.bfloat16),
                w_ref[...].astype(jnp.bfloat16),
                (((1,), (1,)), ((), ())),
                preferred_element_type=jnp.float32)
            if j < 2:
                h = jnp.maximum(h, 0.0)
        h_scratch[...] = h


def kernel(x, adjs, W1, W2, W3):
    last = _N - 1
    return pl.pallas_call(
        _fused_sage_body,
        grid=(_N + 1,),
        in_specs=[
            pl.BlockSpec((_S, _F), lambda i: (jnp.minimum(i, last), 0)),
            pl.BlockSpec((1, _S, _S), lambda i: (jnp.minimum(i, last), 0, 0)),
            pl.BlockSpec((_F, _F), lambda i: (0, 0)),
            pl.BlockSpec((_F, _F), lambda i: (0, 0)),
            pl.BlockSpec((_F, _F), lambda i: (0, 0)),
        ],
        out_specs=pl.BlockSpec((_S, _F), lambda i: (jnp.maximum(i - 1, 0), 0)),
        out_shape=jax.ShapeDtypeStruct(x.shape, x.dtype),
        scratch_shapes=[pltpu.VMEM((_S, _F), jnp.float32)],
        compiler_params=pltpu.CompilerParams(
            dimension_semantics=("arbitrary",)),
    )(x, adjs, W1, W2, W3)


# speedup vs baseline: 1.0589x; 1.0589x over previous
import jax
import jax.numpy as jnp
from jax.experimental import pallas as pl
from jax.experimental.pallas import tpu as pltpu

_S = 1024
_F = 512
_BLOCKS_PER_STEP = 2


def _fused_sage_body(x_ref, adj_ref, w1_ref, w2_ref, w3_ref, out_ref):
    adjs = [adj_ref[b].astype(jnp.bfloat16) for b in range(_BLOCKS_PER_STEP)]
    hs = [x_ref[b * _S:(b + 1) * _S, :] for b in range(_BLOCKS_PER_STEP)]
    for i, w_ref in enumerate((w1_ref, w2_ref, w3_ref)):
        w = w_ref[...].astype(jnp.bfloat16)
        for b in range(_BLOCKS_PER_STEP):
            ax = jnp.dot(adjs[b], hs[b].astype(jnp.bfloat16),
                         preferred_element_type=jnp.float32)
            h = jax.lax.dot_general(
                (hs[b] + ax).astype(jnp.bfloat16), w,
                (((1,), (1,)), ((), ())),
                preferred_element_type=jnp.float32)
            if i < 2:
                h = jnp.maximum(h, 0.0)
            hs[b] = h
    for b in range(_BLOCKS_PER_STEP):
        h = hs[b]
        m = jnp.max(h, axis=1, keepdims=True)
        lse = jnp.log(jnp.sum(jnp.exp(h - m), axis=1, keepdims=True)) + m
        out_ref[b * _S:(b + 1) * _S, :] = h - lse


def kernel(x, adjs, W1, W2, W3):
    nsteps = adjs.shape[0] // _BLOCKS_PER_STEP
    return pl.pallas_call(
        _fused_sage_body,
        grid=(nsteps,),
        in_specs=[
            pl.BlockSpec((_BLOCKS_PER_STEP * _S, _F), lambda i: (i, 0)),
            pl.BlockSpec((_BLOCKS_PER_STEP, _S, _S), lambda i: (i, 0, 0)),
            pl.BlockSpec((_F, _F), lambda i: (0, 0)),
            pl.BlockSpec((_F, _F), lambda i: (0, 0)),
            pl.BlockSpec((_F, _F), lambda i: (0, 0)),
        ],
        out_specs=pl.BlockSpec((_BLOCKS_PER_STEP * _S, _F), lambda i: (i, 0)),
        out_shape=jax.ShapeDtypeStruct(x.shape, x.dtype),
        compiler_params=pltpu.CompilerParams(
            dimension_semantics=("parallel",)),
    )(x, adjs, W1, W2, W3)


# speedup vs baseline: 1.0589x; 1.0000x over previous
import jax
import jax.numpy as jnp
from jax.experimental import pallas as pl
from jax.experimental.pallas import tpu as pltpu

_S = 1024
_F = 512
_BLOCKS_PER_STEP = 2


def _fused_sage_body(x_ref, adj_ref, w1_ref, w2_ref, w3_ref, out_ref):
    adjs = [adj_ref[b].astype(jnp.bfloat16) for b in range(_BLOCKS_PER_STEP)]
    hs = [x_ref[b * _S:(b + 1) * _S, :] for b in range(_BLOCKS_PER_STEP)]
    for i, w_ref in enumerate((w1_ref, w2_ref, w3_ref)):
        w = w_ref[...].astype(jnp.bfloat16)
        for b in range(_BLOCKS_PER_STEP):
            hb = hs[b].astype(jnp.bfloat16)
            ax = (jnp.dot(adjs[b][:, :_F], hb[:_F, :],
                          preferred_element_type=jnp.float32) +
                  jnp.dot(adjs[b][:, _F:], hb[_F:, :],
                          preferred_element_type=jnp.float32))
            h = jax.lax.dot_general(
                (hs[b] + ax).astype(jnp.bfloat16), w,
                (((1,), (1,)), ((), ())),
                preferred_element_type=jnp.float32)
            if i < 2:
                h = jnp.maximum(h, 0.0)
            hs[b] = h
    for b in range(_BLOCKS_PER_STEP):
        h = hs[b]
        m = jnp.max(h, axis=1, keepdims=True)
        lse = jnp.log(jnp.sum(jnp.exp(h - m), axis=1, keepdims=True)) + m
        out_ref[b * _S:(b + 1) * _S, :] = h - lse


def kernel(x, adjs, W1, W2, W3):
    nsteps = adjs.shape[0] // _BLOCKS_PER_STEP
    return pl.pallas_call(
        _fused_sage_body,
        grid=(nsteps,),
        in_specs=[
            pl.BlockSpec((_BLOCKS_PER_STEP * _S, _F), lambda i: (i, 0)),
            pl.BlockSpec((_BLOCKS_PER_STEP, _S, _S), lambda i: (i, 0, 0)),
            pl.BlockSpec((_F, _F), lambda i: (0, 0)),
            pl.BlockSpec((_F, _F), lambda i: (0, 0)),
            pl.BlockSpec((_F, _F), lambda i: (0, 0)),
        ],
        out_specs=pl.BlockSpec((_BLOCKS_PER_STEP * _S, _F), lambda i: (i, 0)),
        out_shape=jax.ShapeDtypeStruct(x.shape, x.dtype),
        compiler_params=pltpu.CompilerParams(
            dimension_semantics=("parallel",)),
    )(x, adjs, W1, W2, W3)
